# baseline (speedup 1.0000x reference)
import functools

import jax
import jax.numpy as jnp
from jax import lax
from jax.experimental import pallas as pl
from jax.experimental.pallas import tpu as pltpu
from jax.experimental.pallas import tpu_sc as plsc

_L = 16
_NBUF = 3


def _sc_body(nrows, w, ch, in_hbm, out_hbm, in_buf, out_buf, in_sem, out_sem):
    info = plsc.get_sparse_core_info()
    nw = info.num_cores * info.num_subcores
    wid = lax.axis_index("s") * info.num_cores + lax.axis_index("c")
    rows_per_w = nrows // nw
    nchunk = rows_per_w // ch
    base = wid * rows_per_w
    half = lax.shift_right_logical(lax.iota(jnp.int32, _L), 1)
    nj = w // _L

    def in_copy(c, slot):
        return pltpu.make_async_copy(
            in_hbm.at[pl.ds(base + c * ch, ch)], in_buf.at[slot], in_sem.at[slot])

    def out_copy(c, slot):
        return pltpu.make_async_copy(
            out_buf.at[slot], out_hbm.at[pl.ds(2 * (base + c * ch), 2 * ch)],
            out_sem.at[slot])

    def compute(slot):
        ib = in_buf.at[slot]
        ob = out_buf.at[slot]

        @functools.partial(plsc.parallel_loop, 0, ch, unroll=4)
        def row_body(r):
            rv = jnp.full((_L,), r, jnp.int32)
            for j in range(nj):
                lo = plsc.load_gather(ib, [rv, _L * j + half])
                hi = plsc.load_gather(ib, [rv, _L * j + 8 + half])
                ob[2 * r, pl.ds(2 * _L * j, _L)] = lo
                ob[2 * r, pl.ds(2 * _L * j + _L, _L)] = hi
                ob[2 * r + 1, pl.ds(2 * _L * j, _L)] = lo
                ob[2 * r + 1, pl.ds(2 * _L * j + _L, _L)] = hi

    for s in range(_NBUF):
        in_copy(s, s).start()

    def group_body(g, _):
        c0 = g * _NBUF
        for s in range(_NBUF):
            c = c0 + s
            in_copy(c, s).wait()
            @pl.when(c >= _NBUF)
            def _():
                out_copy(c - _NBUF, s).wait()
            compute(s)
            out_copy(c, s).start()
            @pl.when(c + _NBUF < nchunk)
            def _():
                in_copy(c + _NBUF, s).start()
        return 0

    lax.fori_loop(0, nchunk // _NBUF, group_body, 0)
    for s in range(_NBUF):
        out_copy(nchunk - _NBUF + s, s).wait()


def kernel(x):
    n, c, h, w = x.shape
    nrows = n * c * h
    ch = 32
    xf = x.reshape(nrows, w)
    body = functools.partial(_sc_body, nrows, w, ch)
    f = pl.kernel(
        body,
        out_type=jax.ShapeDtypeStruct((2 * nrows, 2 * w), x.dtype),
        mesh=plsc.VectorSubcoreMesh(core_axis_name="c", subcore_axis_name="s"),
        compiler_params=pltpu.CompilerParams(needs_layout_passes=False),
        scratch_types=[
            pltpu.VMEM((_NBUF, ch, w), x.dtype),
            pltpu.VMEM((_NBUF, 2 * ch, 2 * w), x.dtype),
            pltpu.SemaphoreType.DMA((_NBUF,)),
            pltpu.SemaphoreType.DMA((_NBUF,)),
        ],
    )
    out = f(xf)
    return out.reshape(n, c, 2 * h, 2 * w)


# speedup vs baseline: 1.0503x; 1.0503x over previous
import functools

import jax
import jax.numpy as jnp
from jax import lax
from jax.experimental import pallas as pl
from jax.experimental.pallas import tpu as pltpu
from jax.experimental.pallas import tpu_sc as plsc

_L = 16
_NBUF = 2


def _sc_body(nrows, w, ch, in_hbm, out_hbm, in_buf, out_buf, in_sem, out_sem):
    info = plsc.get_sparse_core_info()
    nw = info.num_cores * info.num_subcores
    wid = lax.axis_index("s") * info.num_cores + lax.axis_index("c")
    rows_per_w = nrows // nw
    nchunk = rows_per_w // ch
    base = wid * rows_per_w
    half = lax.shift_right_logical(lax.iota(jnp.int32, _L), 1)
    nj = w // _L

    def in_copy(c, slot):
        return pltpu.make_async_copy(
            in_hbm.at[pl.ds(base + c * ch, ch)], in_buf.at[slot], in_sem.at[slot])

    def out_copy(c, slot):
        return pltpu.make_async_copy(
            out_buf.at[slot], out_hbm.at[pl.ds(2 * (base + c * ch), 2 * ch)],
            out_sem.at[slot])

    def compute(slot):
        ib = in_buf.at[slot]
        ob = out_buf.at[slot]

        @functools.partial(plsc.parallel_loop, 0, ch, unroll=4)
        def row_body(r):
            rv = jnp.full((_L,), r, jnp.int32)
            for j in range(nj):
                lo = plsc.load_gather(ib, [rv, _L * j + half])
                hi = plsc.load_gather(ib, [rv, _L * j + 8 + half])
                ob[2 * r, pl.ds(2 * _L * j, _L)] = lo
                ob[2 * r, pl.ds(2 * _L * j + _L, _L)] = hi
                ob[2 * r + 1, pl.ds(2 * _L * j, _L)] = lo
                ob[2 * r + 1, pl.ds(2 * _L * j + _L, _L)] = hi

    for s in range(_NBUF):
        in_copy(s, s).start()

    def group_body(g, _):
        c0 = g * _NBUF
        for s in range(_NBUF):
            c = c0 + s
            in_copy(c, s).wait()
            @pl.when(c >= _NBUF)
            def _():
                out_copy(c - _NBUF, s).wait()
            compute(s)
            out_copy(c, s).start()
            @pl.when(c + _NBUF < nchunk)
            def _():
                in_copy(c + _NBUF, s).start()
        return 0

    lax.fori_loop(0, nchunk // _NBUF, group_body, 0)
    for s in range(_NBUF):
        out_copy(nchunk - _NBUF + s, s).wait()


def kernel(x):
    n, c, h, w = x.shape
    nrows = n * c * h
    ch = 64
    xf = x.reshape(nrows, w)
    body = functools.partial(_sc_body, nrows, w, ch)
    f = pl.kernel(
        body,
        out_type=jax.ShapeDtypeStruct((2 * nrows, 2 * w), x.dtype),
        mesh=plsc.VectorSubcoreMesh(core_axis_name="c", subcore_axis_name="s"),
        compiler_params=pltpu.CompilerParams(needs_layout_passes=False),
        scratch_types=[
            pltpu.VMEM((_NBUF, ch, w), x.dtype),
            pltpu.VMEM((_NBUF, 2 * ch, 2 * w), x.dtype),
            pltpu.SemaphoreType.DMA((_NBUF,)),
            pltpu.SemaphoreType.DMA((_NBUF,)),
        ],
    )
    out = f(xf)
    return out.reshape(n, c, 2 * h, 2 * w)
